# Initial kernel scaffold; baseline (speedup 1.0000x reference)
import jax
import jax.numpy as jnp
from jax.experimental import pallas as pl


def kernel(Z, element_embedding, W, electron_config):
    raise NotImplementedError("write your pallas kernel here")


# speedup vs baseline: 1.0221x; 1.0221x over previous
import functools

import jax
import jax.numpy as jnp
from jax import lax
from jax.experimental import pallas as pl
from jax.experimental.pallas import tpu as pltpu
from jax.experimental.pallas import tpu_sc as plsc

NUM_FEATURES = 512
ZMAX = 87
CONFIG_DIM = 20
N_ATOMS = 100000

NC = 2
NS = 16
NW = NC * NS

CHUNK = 80
NCH = N_ATOMS // CHUNK
T_MAX = (NCH + NW - 1) // NW


def _table_body(ee_ref, ec_ref, w_ref, out_ref):
    out_ref[...] = ee_ref[...] + lax.dot_general(
        ec_ref[...], w_ref[...],
        dimension_numbers=(((1,), (1,)), ((), ())),
        preferred_element_type=jnp.float32,
    )


def _build_table(element_embedding, electron_config, W):
    return pl.pallas_call(
        _table_body,
        out_shape=jax.ShapeDtypeStruct((ZMAX, NUM_FEATURES), jnp.float32),
    )(element_embedding, electron_config, W)


_mesh = plsc.VectorSubcoreMesh(
    core_axis_name="c", subcore_axis_name="s", num_cores=NC, num_subcores=NS
)


@functools.partial(
    pl.kernel,
    out_type=jax.ShapeDtypeStruct((N_ATOMS, NUM_FEATURES), jnp.float32),
    mesh=_mesh,
    scratch_types=[
        pltpu.VMEM((CHUNK,), jnp.int32),
        pltpu.VMEM((CHUNK, NUM_FEATURES), jnp.float32),
        pltpu.SemaphoreType.DMA,
    ],
)
def _gather_kernel(table_hbm, z_hbm, out_hbm, idx_v, rows_v, sem):
    wid = lax.axis_index("s") * NC + lax.axis_index("c")

    def body(t, carry):
        cid = wid + t * NW

        @pl.when(cid < NCH)
        def _():
            base = cid * CHUNK
            pltpu.sync_copy(z_hbm.at[pl.ds(base, CHUNK)], idx_v)
            pltpu.async_copy(table_hbm.at[idx_v], rows_v, sem).wait()
            pltpu.sync_copy(rows_v, out_hbm.at[pl.ds(base, CHUNK)])

        return carry

    lax.fori_loop(0, T_MAX, body, 0)


def kernel(Z, element_embedding, W, electron_config):
    table = _build_table(element_embedding, electron_config, W)
    return _gather_kernel(table, Z.astype(jnp.int32))


# speedup vs baseline: 1.0232x; 1.0011x over previous
import functools

import jax
import jax.numpy as jnp
from jax import lax
from jax.experimental import pallas as pl
from jax.experimental.pallas import tpu as pltpu
from jax.experimental.pallas import tpu_sc as plsc

NUM_FEATURES = 512
ZMAX = 87
CONFIG_DIM = 20
N_ATOMS = 100000

NC = 2
NS = 16
NW = NC * NS

CHUNK = 80
NCH = N_ATOMS // CHUNK
T_MAX = (NCH + NW - 1) // NW
S_MAX = T_MAX // 2


def _table_body(ee_ref, ec_ref, w_ref, out_ref):
    out_ref[...] = ee_ref[...] + lax.dot_general(
        ec_ref[...], w_ref[...],
        dimension_numbers=(((1,), (1,)), ((), ())),
        preferred_element_type=jnp.float32,
    )


def _build_table(element_embedding, electron_config, W):
    return pl.pallas_call(
        _table_body,
        out_shape=jax.ShapeDtypeStruct((ZMAX, NUM_FEATURES), jnp.float32),
    )(element_embedding, electron_config, W)


_mesh = plsc.VectorSubcoreMesh(
    core_axis_name="c", subcore_axis_name="s", num_cores=NC, num_subcores=NS
)


@functools.partial(
    pl.kernel,
    out_type=jax.ShapeDtypeStruct((N_ATOMS, NUM_FEATURES), jnp.float32),
    mesh=_mesh,
    scratch_types=[
        pltpu.VMEM((2, CHUNK), jnp.int32),
        pltpu.VMEM((CHUNK, NUM_FEATURES), jnp.float32),
        pltpu.VMEM((CHUNK, NUM_FEATURES), jnp.float32),
        pltpu.SemaphoreType.DMA,
        pltpu.SemaphoreType.DMA,
        pltpu.SemaphoreType.DMA,
        pltpu.SemaphoreType.DMA,
        pltpu.SemaphoreType.DMA,
        pltpu.SemaphoreType.DMA,
    ],
)
def _gather_kernel(table_hbm, z_hbm, out_hbm, idx_v, rows0, rows1, si0, si1,
                   sg0, sg1, so0, so1):
    wid = lax.axis_index("s") * NC + lax.axis_index("c")
    rows = (rows0, rows1)
    si = (si0, si1)
    sg = (sg0, sg1)
    so = (so0, so1)

    def cid_of(tt):
        return wid + tt * NW

    def idx_start(tt, b):
        @pl.when(cid_of(tt) < NCH)
        def _():
            pltpu.async_copy(
                z_hbm.at[pl.ds(cid_of(tt) * CHUNK, CHUNK)], idx_v.at[b], si[b]
            )

    def idx_wait(b):
        pltpu.make_async_copy(
            z_hbm.at[pl.ds(0, CHUNK)], idx_v.at[b], si[b]
        ).wait()

    def gather_wait(b):
        pltpu.make_async_copy(
            table_hbm.at[idx_v.at[b]], rows[b], sg[b]
        ).wait()

    def out_wait(b):
        pltpu.make_async_copy(
            rows[b], out_hbm.at[pl.ds(0, CHUNK)], so[b]
        ).wait()

    idx_start(0, 0)
    idx_start(1, 1)

    @pl.when(cid_of(0) < NCH)
    def _():
        idx_wait(0)
        pltpu.async_copy(table_hbm.at[idx_v.at[0]], rows[0], sg[0])

    def body(s, carry):
        for b in (0, 1):
            tt = 2 * s + b
            bo = 1 - b

            @pl.when(cid_of(tt + 1) < NCH)
            def _():
                @pl.when(tt + 1 >= 2)
                def _():
                    out_wait(bo)

                idx_wait(bo)
                pltpu.async_copy(table_hbm.at[idx_v.at[bo]], rows[bo], sg[bo])

            @pl.when(cid_of(tt) < NCH)
            def _():
                gather_wait(b)
                pltpu.async_copy(
                    rows[b], out_hbm.at[pl.ds(cid_of(tt) * CHUNK, CHUNK)],
                    so[b],
                )

            idx_start(tt + 2, b)
        return carry

    lax.fori_loop(0, S_MAX, body, 0)

    out_wait(0)
    out_wait(1)


def kernel(Z, element_embedding, W, electron_config):
    table = _build_table(element_embedding, electron_config, W)
    return _gather_kernel(table, Z.astype(jnp.int32))
